# speedup vs baseline: 4.0489x; 4.0489x over previous
import functools

import jax
import jax.numpy as jnp
from jax.experimental import pallas as pl

_G = 128
_BLK = 512


def _main_body(x_ref, b_ref, w1_ref, b1_ref, w2_ref, b2_ref, wf_ref, bf_ref,
               out_ref, m_ref, d_ref, s_ref, *, nb: int):
    i = pl.program_id(0)

    @pl.when(i == 0)
    def _init():
        out_ref[...] = jnp.zeros_like(out_ref)
        m_ref[...] = jnp.full_like(m_ref, -jnp.inf)
        d_ref[...] = jnp.zeros_like(d_ref)

    xb = x_ref[...]
    h = jnp.maximum(jnp.dot(xb, w1_ref[...],
                            preferred_element_type=jnp.float32)
                    + b1_ref[...], 0.0)
    s = jnp.sum(h * w2_ref[...], axis=1, keepdims=True) + b2_ref[...]
    s_ref[...] = s

    bid = b_ref[...]
    gi = jax.lax.broadcasted_iota(jnp.int32, (bid.shape[0], _G), 1)
    bmask = bid == gi

    blockmax = jnp.max(jnp.where(bmask, s, -jnp.inf), axis=0, keepdims=True)
    m_old = m_ref[...]
    m_new = jnp.maximum(m_old, blockmax)
    scale = jnp.where(m_new == -jnp.inf, 1.0, jnp.exp(m_old - m_new))
    m_ref[...] = m_new

    mg = jnp.sum(jnp.where(bmask, m_new, 0.0), axis=1, keepdims=True)
    p = jnp.exp(s - mg)
    w = jnp.where(bmask, p, 0.0)
    d_ref[...] = d_ref[...] * scale + jnp.sum(w, axis=0, keepdims=True)

    f = jnp.maximum(jnp.dot(xb, wf_ref[...],
                            preferred_element_type=jnp.float32)
                    + bf_ref[...], 0.0)
    contrib = jax.lax.dot_general(f, w, (((0,), (0,)), ((), ())),
                                  preferred_element_type=jnp.float32)
    out_ref[...] = out_ref[...] * scale + contrib

    @pl.when(i == nb - 1)
    def _fin():
        d_fin = d_ref[...]
        out_ref[...] = jnp.where(d_fin > 0.0, out_ref[...] / d_fin, 0.0)


def _alpha_body(s_ref, b_ref, m_ref, d_ref, a_ref):
    s = s_ref[...]
    bid = b_ref[...]
    gi = jax.lax.broadcasted_iota(jnp.int32, (bid.shape[0], _G), 1)
    bmask = bid == gi
    mg = jnp.sum(jnp.where(bmask, m_ref[...], 0.0), axis=1, keepdims=True)
    dg = jnp.sum(jnp.where(bmask, d_ref[...], 0.0), axis=1, keepdims=True)
    a_ref[...] = jnp.exp(s - mg) / jnp.where(dg > 0.0, dg, 1.0)


def kernel(x, batch, W1g, b1g, W2g, b2g, Wf, bf):
    n, d_model = x.shape
    h_dim = W1g.shape[1]
    nb = (n + _BLK - 1) // _BLK
    npad = nb * _BLK

    x_p = jnp.pad(x, ((0, npad - n), (0, 0)))
    batch_p = jnp.pad(batch, (0, npad - n), constant_values=_G)
    batch2d = batch_p.reshape(npad, 1)
    b1r = b1g.reshape(1, h_dim)
    w2r = W2g.reshape(1, h_dim)
    b2r = b2g.reshape(1, 1)
    bfr = bf.reshape(1, d_model)

    const = lambda i: (0, 0)
    row = lambda i: (i, 0)

    out_t, m, d, scores = pl.pallas_call(
        functools.partial(_main_body, nb=nb),
        grid=(nb,),
        in_specs=[
            pl.BlockSpec((_BLK, d_model), row),
            pl.BlockSpec((_BLK, 1), row),
            pl.BlockSpec((d_model, h_dim), const),
            pl.BlockSpec((1, h_dim), const),
            pl.BlockSpec((1, h_dim), const),
            pl.BlockSpec((1, 1), const),
            pl.BlockSpec((d_model, d_model), const),
            pl.BlockSpec((1, d_model), const),
        ],
        out_specs=[
            pl.BlockSpec((d_model, _G), const),
            pl.BlockSpec((1, _G), const),
            pl.BlockSpec((1, _G), const),
            pl.BlockSpec((_BLK, 1), row),
        ],
        out_shape=[
            jax.ShapeDtypeStruct((d_model, _G), jnp.float32),
            jax.ShapeDtypeStruct((1, _G), jnp.float32),
            jax.ShapeDtypeStruct((1, _G), jnp.float32),
            jax.ShapeDtypeStruct((npad, 1), jnp.float32),
        ],
    )(x_p, batch2d, W1g, b1r, w2r, b2r, Wf, bfr)

    alpha = pl.pallas_call(
        _alpha_body,
        grid=(nb,),
        in_specs=[
            pl.BlockSpec((_BLK, 1), row),
            pl.BlockSpec((_BLK, 1), row),
            pl.BlockSpec((1, _G), const),
            pl.BlockSpec((1, _G), const),
        ],
        out_specs=pl.BlockSpec((_BLK, 1), row),
        out_shape=jax.ShapeDtypeStruct((npad, 1), jnp.float32),
    )(scores, batch2d, m, d)

    return out_t.T, alpha[:n, 0]


# speedup vs baseline: 4.7277x; 1.1676x over previous
import functools

import jax
import jax.numpy as jnp
from jax.experimental import pallas as pl

_G = 128
_BLK = 512


def _main_body(x_ref, b_ref, w1_ref, b1_ref, w2_ref, b2_ref, wf_ref, bf_ref,
               out_ref, m_ref, d_ref, s_ref, *, nb: int, n: int):
    i = pl.program_id(0)

    @pl.when(i == 0)
    def _init():
        out_ref[...] = jnp.zeros_like(out_ref)
        m_ref[...] = jnp.full_like(m_ref, -jnp.inf)
        d_ref[...] = jnp.zeros_like(d_ref)

    blk = x_ref.shape[0]
    ridx = i * blk + jax.lax.broadcasted_iota(jnp.int32, (blk, 1), 0)
    valid = ridx < n
    xb = jnp.where(valid, x_ref[...], 0.0)
    h = jnp.maximum(jnp.dot(xb, w1_ref[...],
                            preferred_element_type=jnp.float32)
                    + b1_ref[...], 0.0)
    s = jnp.sum(h * w2_ref[...], axis=1, keepdims=True) + b2_ref[...]
    s_ref[...] = s

    bid = b_ref[...]
    gi = jax.lax.broadcasted_iota(jnp.int32, (blk, _G), 1)
    bmask = (bid == gi) & valid

    blockmax = jnp.max(jnp.where(bmask, s, -jnp.inf), axis=0, keepdims=True)
    m_old = m_ref[...]
    m_new = jnp.maximum(m_old, blockmax)
    scale = jnp.where(m_new == -jnp.inf, 1.0, jnp.exp(m_old - m_new))
    m_ref[...] = m_new

    mg = jnp.sum(jnp.where(bmask, m_new, 0.0), axis=1, keepdims=True)
    p = jnp.exp(s - mg)
    w = jnp.where(bmask, p, 0.0)
    d_ref[...] = d_ref[...] * scale + jnp.sum(w, axis=0, keepdims=True)

    f = jnp.maximum(jnp.dot(xb, wf_ref[...],
                            preferred_element_type=jnp.float32)
                    + bf_ref[...], 0.0)
    contrib = jax.lax.dot_general(f, w, (((0,), (0,)), ((), ())),
                                  preferred_element_type=jnp.float32)
    out_ref[...] = out_ref[...] * scale + contrib

    @pl.when(i == nb - 1)
    def _fin():
        d_fin = d_ref[...]
        out_ref[...] = jnp.where(d_fin > 0.0, out_ref[...] / d_fin, 0.0)


def _alpha_body(s_ref, b_ref, m_ref, d_ref, a_ref):
    s = s_ref[...]
    bid = b_ref[...]
    gi = jax.lax.broadcasted_iota(jnp.int32, (bid.shape[0], _G), 1)
    bmask = bid == gi
    mg = jnp.sum(jnp.where(bmask, m_ref[...], 0.0), axis=1, keepdims=True)
    dg = jnp.sum(jnp.where(bmask, d_ref[...], 0.0), axis=1, keepdims=True)
    a_ref[...] = jnp.exp(s - mg) / jnp.where(dg > 0.0, dg, 1.0)


def kernel(x, batch, W1g, b1g, W2g, b2g, Wf, bf):
    n, d_model = x.shape
    h_dim = W1g.shape[1]
    nb = (n + _BLK - 1) // _BLK

    batch2d = batch.reshape(n, 1)
    b1r = b1g.reshape(1, h_dim)
    w2r = W2g.reshape(1, h_dim)
    b2r = b2g.reshape(1, 1)
    bfr = bf.reshape(1, d_model)

    const = lambda i: (0, 0)
    row = lambda i: (i, 0)

    out_t, m, d, scores = pl.pallas_call(
        functools.partial(_main_body, nb=nb, n=n),
        grid=(nb,),
        in_specs=[
            pl.BlockSpec((_BLK, d_model), row),
            pl.BlockSpec((_BLK, 1), row),
            pl.BlockSpec((d_model, h_dim), const),
            pl.BlockSpec((1, h_dim), const),
            pl.BlockSpec((1, h_dim), const),
            pl.BlockSpec((1, 1), const),
            pl.BlockSpec((d_model, d_model), const),
            pl.BlockSpec((1, d_model), const),
        ],
        out_specs=[
            pl.BlockSpec((d_model, _G), const),
            pl.BlockSpec((1, _G), const),
            pl.BlockSpec((1, _G), const),
            pl.BlockSpec((_BLK, 1), row),
        ],
        out_shape=[
            jax.ShapeDtypeStruct((d_model, _G), jnp.float32),
            jax.ShapeDtypeStruct((1, _G), jnp.float32),
            jax.ShapeDtypeStruct((1, _G), jnp.float32),
            jax.ShapeDtypeStruct((n, 1), jnp.float32),
        ],
    )(x, batch2d, W1g, b1r, w2r, b2r, Wf, bfr)

    alpha = pl.pallas_call(
        _alpha_body,
        grid=(nb,),
        in_specs=[
            pl.BlockSpec((_BLK, 1), row),
            pl.BlockSpec((_BLK, 1), row),
            pl.BlockSpec((1, _G), const),
            pl.BlockSpec((1, _G), const),
        ],
        out_specs=pl.BlockSpec((_BLK, 1), row),
        out_shape=jax.ShapeDtypeStruct((n, 1), jnp.float32),
    )(scores, batch2d, m, d)

    return out_t.T, alpha[:, 0]


# speedup vs baseline: 5.9048x; 1.2490x over previous
import functools

import jax
import jax.numpy as jnp
from jax.experimental import pallas as pl

_G = 128
_BLK = 1024


def _main_body(x_ref, b_ref, w1_ref, b1_ref, w2_ref, b2_ref, wf_ref, bf_ref,
               out_ref, m_ref, d_ref, s_ref, *, nb: int, n: int):
    i = pl.program_id(0)

    @pl.when(i == 0)
    def _init():
        out_ref[...] = jnp.zeros_like(out_ref)
        m_ref[...] = jnp.full_like(m_ref, -jnp.inf)
        d_ref[...] = jnp.zeros_like(d_ref)

    blk = x_ref.shape[0]
    ridx = i * blk + jax.lax.broadcasted_iota(jnp.int32, (blk, 1), 0)
    valid = ridx < n
    xb = jnp.where(valid, x_ref[...], 0.0)
    h = jnp.maximum(jnp.dot(xb, w1_ref[...],
                            preferred_element_type=jnp.float32)
                    + b1_ref[...], 0.0)
    s = jnp.sum(h * w2_ref[...], axis=1, keepdims=True) + b2_ref[...]
    s_ref[...] = s

    bid = b_ref[...]
    gi = jax.lax.broadcasted_iota(jnp.int32, (blk, _G), 1)
    bmask = (bid == gi) & valid

    blockmax = jnp.max(jnp.where(bmask, s, -jnp.inf), axis=0, keepdims=True)
    m_old = m_ref[...]
    m_new = jnp.maximum(m_old, blockmax)
    scale = jnp.where(m_new == -jnp.inf, 1.0, jnp.exp(m_old - m_new))
    m_ref[...] = m_new

    mg = jnp.sum(jnp.where(bmask, m_new, 0.0), axis=1, keepdims=True)
    p = jnp.exp(s - mg)
    w = jnp.where(bmask, p, 0.0)
    d_ref[...] = d_ref[...] * scale + jnp.sum(w, axis=0, keepdims=True)

    f = jnp.maximum(jnp.dot(xb, wf_ref[...],
                            preferred_element_type=jnp.float32)
                    + bf_ref[...], 0.0)
    contrib = jax.lax.dot_general(f, w, (((0,), (0,)), ((), ())),
                                  preferred_element_type=jnp.float32)
    out_ref[...] = out_ref[...] * scale + contrib

    @pl.when(i == nb - 1)
    def _fin():
        d_fin = d_ref[...]
        out_ref[...] = jnp.where(d_fin > 0.0, out_ref[...] / d_fin, 0.0)


def _alpha_body(s_ref, b_ref, m_ref, d_ref, a_ref):
    s = s_ref[...]
    bid = b_ref[...]
    gi = jax.lax.broadcasted_iota(jnp.int32, (bid.shape[0], _G), 1)
    bmask = bid == gi
    mg = jnp.sum(jnp.where(bmask, m_ref[...], 0.0), axis=1, keepdims=True)
    dg = jnp.sum(jnp.where(bmask, d_ref[...], 0.0), axis=1, keepdims=True)
    a_ref[...] = jnp.exp(s - mg) / jnp.where(dg > 0.0, dg, 1.0)


def kernel(x, batch, W1g, b1g, W2g, b2g, Wf, bf):
    n, d_model = x.shape
    h_dim = W1g.shape[1]
    nb = (n + _BLK - 1) // _BLK

    batch2d = batch.reshape(n, 1)
    b1r = b1g.reshape(1, h_dim)
    w2r = W2g.reshape(1, h_dim)
    b2r = b2g.reshape(1, 1)
    bfr = bf.reshape(1, d_model)

    const = lambda i: (0, 0)
    row = lambda i: (i, 0)

    out_t, m, d, scores = pl.pallas_call(
        functools.partial(_main_body, nb=nb, n=n),
        grid=(nb,),
        in_specs=[
            pl.BlockSpec((_BLK, d_model), row),
            pl.BlockSpec((_BLK, 1), row),
            pl.BlockSpec((d_model, h_dim), const),
            pl.BlockSpec((1, h_dim), const),
            pl.BlockSpec((1, h_dim), const),
            pl.BlockSpec((1, 1), const),
            pl.BlockSpec((d_model, d_model), const),
            pl.BlockSpec((1, d_model), const),
        ],
        out_specs=[
            pl.BlockSpec((d_model, _G), const),
            pl.BlockSpec((1, _G), const),
            pl.BlockSpec((1, _G), const),
            pl.BlockSpec((_BLK, 1), row),
        ],
        out_shape=[
            jax.ShapeDtypeStruct((d_model, _G), jnp.float32),
            jax.ShapeDtypeStruct((1, _G), jnp.float32),
            jax.ShapeDtypeStruct((1, _G), jnp.float32),
            jax.ShapeDtypeStruct((n, 1), jnp.float32),
        ],
    )(x, batch2d, W1g, b1r, w2r, b2r, Wf, bfr)

    alpha = pl.pallas_call(
        _alpha_body,
        grid=(nb,),
        in_specs=[
            pl.BlockSpec((_BLK, 1), row),
            pl.BlockSpec((_BLK, 1), row),
            pl.BlockSpec((1, _G), const),
            pl.BlockSpec((1, _G), const),
        ],
        out_specs=pl.BlockSpec((_BLK, 1), row),
        out_shape=jax.ShapeDtypeStruct((n, 1), jnp.float32),
    )(scores, batch2d, m, d)

    return out_t.T, alpha[:, 0]


# speedup vs baseline: 6.1712x; 1.0451x over previous
import functools

import jax
import jax.numpy as jnp
from jax.experimental import pallas as pl

_G = 128
_BLK = 2048


def _main_body(x_ref, b_ref, w1_ref, b1_ref, w2_ref, b2_ref, wf_ref, bf_ref,
               out_ref, m_ref, d_ref, s_ref, *, nb: int, n: int):
    i = pl.program_id(0)

    @pl.when(i == 0)
    def _init():
        out_ref[...] = jnp.zeros_like(out_ref)
        m_ref[...] = jnp.full_like(m_ref, -jnp.inf)
        d_ref[...] = jnp.zeros_like(d_ref)

    blk = x_ref.shape[0]
    ridx = i * blk + jax.lax.broadcasted_iota(jnp.int32, (blk, 1), 0)
    valid = ridx < n
    xb = jnp.where(valid, x_ref[...], 0.0)
    h = jnp.maximum(jnp.dot(xb, w1_ref[...],
                            preferred_element_type=jnp.float32)
                    + b1_ref[...], 0.0)
    s = jnp.sum(h * w2_ref[...], axis=1, keepdims=True) + b2_ref[...]
    s_ref[...] = s

    bid = b_ref[...]
    gi = jax.lax.broadcasted_iota(jnp.int32, (blk, _G), 1)
    bmask = (bid == gi) & valid

    blockmax = jnp.max(jnp.where(bmask, s, -jnp.inf), axis=0, keepdims=True)
    m_old = m_ref[...]
    m_new = jnp.maximum(m_old, blockmax)
    scale = jnp.where(m_new == -jnp.inf, 1.0, jnp.exp(m_old - m_new))
    m_ref[...] = m_new

    mg = jnp.sum(jnp.where(bmask, m_new, 0.0), axis=1, keepdims=True)
    p = jnp.exp(s - mg)
    w = jnp.where(bmask, p, 0.0)
    d_ref[...] = d_ref[...] * scale + jnp.sum(w, axis=0, keepdims=True)

    f = jnp.maximum(jnp.dot(xb, wf_ref[...],
                            preferred_element_type=jnp.float32)
                    + bf_ref[...], 0.0)
    contrib = jax.lax.dot_general(f, w, (((0,), (0,)), ((), ())),
                                  preferred_element_type=jnp.float32)
    out_ref[...] = out_ref[...] * scale + contrib

    @pl.when(i == nb - 1)
    def _fin():
        d_fin = d_ref[...]
        out_ref[...] = jnp.where(d_fin > 0.0, out_ref[...] / d_fin, 0.0)


def _alpha_body(s_ref, b_ref, m_ref, d_ref, a_ref):
    s = s_ref[...]
    bid = b_ref[...]
    gi = jax.lax.broadcasted_iota(jnp.int32, (bid.shape[0], _G), 1)
    bmask = bid == gi
    mg = jnp.sum(jnp.where(bmask, m_ref[...], 0.0), axis=1, keepdims=True)
    dg = jnp.sum(jnp.where(bmask, d_ref[...], 0.0), axis=1, keepdims=True)
    a_ref[...] = jnp.exp(s - mg) / jnp.where(dg > 0.0, dg, 1.0)


def kernel(x, batch, W1g, b1g, W2g, b2g, Wf, bf):
    n, d_model = x.shape
    h_dim = W1g.shape[1]
    nb = (n + _BLK - 1) // _BLK

    batch2d = batch.reshape(n, 1)
    b1r = b1g.reshape(1, h_dim)
    w2r = W2g.reshape(1, h_dim)
    b2r = b2g.reshape(1, 1)
    bfr = bf.reshape(1, d_model)

    const = lambda i: (0, 0)
    row = lambda i: (i, 0)

    out_t, m, d, scores = pl.pallas_call(
        functools.partial(_main_body, nb=nb, n=n),
        grid=(nb,),
        in_specs=[
            pl.BlockSpec((_BLK, d_model), row),
            pl.BlockSpec((_BLK, 1), row),
            pl.BlockSpec((d_model, h_dim), const),
            pl.BlockSpec((1, h_dim), const),
            pl.BlockSpec((1, h_dim), const),
            pl.BlockSpec((1, 1), const),
            pl.BlockSpec((d_model, d_model), const),
            pl.BlockSpec((1, d_model), const),
        ],
        out_specs=[
            pl.BlockSpec((d_model, _G), const),
            pl.BlockSpec((1, _G), const),
            pl.BlockSpec((1, _G), const),
            pl.BlockSpec((_BLK, 1), row),
        ],
        out_shape=[
            jax.ShapeDtypeStruct((d_model, _G), jnp.float32),
            jax.ShapeDtypeStruct((1, _G), jnp.float32),
            jax.ShapeDtypeStruct((1, _G), jnp.float32),
            jax.ShapeDtypeStruct((n, 1), jnp.float32),
        ],
    )(x, batch2d, W1g, b1r, w2r, b2r, Wf, bfr)

    alpha = pl.pallas_call(
        _alpha_body,
        grid=(nb,),
        in_specs=[
            pl.BlockSpec((_BLK, 1), row),
            pl.BlockSpec((_BLK, 1), row),
            pl.BlockSpec((1, _G), const),
            pl.BlockSpec((1, _G), const),
        ],
        out_specs=pl.BlockSpec((_BLK, 1), row),
        out_shape=jax.ShapeDtypeStruct((n, 1), jnp.float32),
    )(scores, batch2d, m, d)

    return out_t.T, alpha[:, 0]


# speedup vs baseline: 6.2495x; 1.0127x over previous
import functools

import jax
import jax.numpy as jnp
from jax.experimental import pallas as pl

_G = 128
_BLK = 2000


def _main_body(x_ref, b_ref, w1_ref, b1_ref, w2_ref, b2_ref, wf_ref, bf_ref,
               out_ref, m_ref, d_ref, s_ref, *, nb: int, n: int):
    i = pl.program_id(0)

    @pl.when(i == 0)
    def _init():
        out_ref[...] = jnp.zeros_like(out_ref)
        m_ref[...] = jnp.full_like(m_ref, -jnp.inf)
        d_ref[...] = jnp.zeros_like(d_ref)

    blk = x_ref.shape[0]
    ridx = i * blk + jax.lax.broadcasted_iota(jnp.int32, (blk, 1), 0)
    valid = ridx < n
    xb = jnp.where(valid, x_ref[...], 0.0)
    h = jnp.maximum(jnp.dot(xb, w1_ref[...],
                            preferred_element_type=jnp.float32)
                    + b1_ref[...], 0.0)
    s = jnp.sum(h * w2_ref[...], axis=1, keepdims=True) + b2_ref[...]
    s_ref[...] = s

    bid = b_ref[...]
    gi = jax.lax.broadcasted_iota(jnp.int32, (blk, _G), 1)
    bmask = (bid == gi) & valid

    blockmax = jnp.max(jnp.where(bmask, s, -jnp.inf), axis=0, keepdims=True)
    m_old = m_ref[...]
    m_new = jnp.maximum(m_old, blockmax)
    scale = jnp.where(m_new == -jnp.inf, 1.0, jnp.exp(m_old - m_new))
    m_ref[...] = m_new

    mg = jnp.sum(jnp.where(bmask, m_new, 0.0), axis=1, keepdims=True)
    p = jnp.exp(s - mg)
    w = jnp.where(bmask, p, 0.0)
    d_ref[...] = d_ref[...] * scale + jnp.sum(w, axis=0, keepdims=True)

    f = jnp.maximum(jnp.dot(xb, wf_ref[...],
                            preferred_element_type=jnp.float32)
                    + bf_ref[...], 0.0)
    contrib = jax.lax.dot_general(f, w, (((0,), (0,)), ((), ())),
                                  preferred_element_type=jnp.float32)
    out_ref[...] = out_ref[...] * scale + contrib

    @pl.when(i == nb - 1)
    def _fin():
        d_fin = d_ref[...]
        out_ref[...] = jnp.where(d_fin > 0.0, out_ref[...] / d_fin, 0.0)


def _alpha_body(s_ref, b_ref, m_ref, d_ref, a_ref):
    s = s_ref[...]
    bid = b_ref[...]
    gi = jax.lax.broadcasted_iota(jnp.int32, (bid.shape[0], _G), 1)
    bmask = bid == gi
    mg = jnp.sum(jnp.where(bmask, m_ref[...], 0.0), axis=1, keepdims=True)
    dg = jnp.sum(jnp.where(bmask, d_ref[...], 0.0), axis=1, keepdims=True)
    a_ref[...] = jnp.exp(s - mg) / jnp.where(dg > 0.0, dg, 1.0)


def kernel(x, batch, W1g, b1g, W2g, b2g, Wf, bf):
    n, d_model = x.shape
    h_dim = W1g.shape[1]
    nb = (n + _BLK - 1) // _BLK

    batch2d = batch.reshape(n, 1)
    b1r = b1g.reshape(1, h_dim)
    w2r = W2g.reshape(1, h_dim)
    b2r = b2g.reshape(1, 1)
    bfr = bf.reshape(1, d_model)

    const = lambda i: (0, 0)
    row = lambda i: (i, 0)

    out_t, m, d, scores = pl.pallas_call(
        functools.partial(_main_body, nb=nb, n=n),
        grid=(nb,),
        in_specs=[
            pl.BlockSpec((_BLK, d_model), row),
            pl.BlockSpec((_BLK, 1), row),
            pl.BlockSpec((d_model, h_dim), const),
            pl.BlockSpec((1, h_dim), const),
            pl.BlockSpec((1, h_dim), const),
            pl.BlockSpec((1, 1), const),
            pl.BlockSpec((d_model, d_model), const),
            pl.BlockSpec((1, d_model), const),
        ],
        out_specs=[
            pl.BlockSpec((d_model, _G), const),
            pl.BlockSpec((1, _G), const),
            pl.BlockSpec((1, _G), const),
            pl.BlockSpec((_BLK, 1), row),
        ],
        out_shape=[
            jax.ShapeDtypeStruct((d_model, _G), jnp.float32),
            jax.ShapeDtypeStruct((1, _G), jnp.float32),
            jax.ShapeDtypeStruct((1, _G), jnp.float32),
            jax.ShapeDtypeStruct((n, 1), jnp.float32),
        ],
    )(x, batch2d, W1g, b1r, w2r, b2r, Wf, bfr)

    alpha = pl.pallas_call(
        _alpha_body,
        grid=(nb,),
        in_specs=[
            pl.BlockSpec((_BLK, 1), row),
            pl.BlockSpec((_BLK, 1), row),
            pl.BlockSpec((1, _G), const),
            pl.BlockSpec((1, _G), const),
        ],
        out_specs=pl.BlockSpec((_BLK, 1), row),
        out_shape=jax.ShapeDtypeStruct((n, 1), jnp.float32),
    )(scores, batch2d, m, d)

    return out_t.T, alpha[:, 0]
